# baseline (speedup 1.0000x reference)
import functools

import jax
import jax.numpy as jnp
from jax import lax
from jax.experimental import pallas as pl
from jax.experimental.pallas import tpu as pltpu
from jax.experimental.pallas import tpu_sc as plsc

N = 10000
E = 320000
DF = 128
G = 64
A_OUT = 32

NC, NS = 2, 16
NW = NC * NS
CB = 128
NBUF = 8
CH = 80
EP = NW * CH * CB
NP = 10240
RPT = NP // NS
DEGW = 1


def _make_edge_scatter(width):
    mesh = plsc.VectorSubcoreMesh(core_axis_name="c", subcore_axis_name="s")

    @functools.partial(
        pl.kernel,
        out_type=jax.ShapeDtypeStruct((NC, NP, width), jnp.float32),
        mesh=mesh,
        compiler_params=pltpu.CompilerParams(use_tc_tiling_on_sc=False),
        scratch_types=(
            [pltpu.VMEM((CH, CB), jnp.int32),
             pltpu.VMEM((CH, CB), jnp.int32),
             pltpu.VMEM_SHARED((NP, width), jnp.float32)]
            + [pltpu.VMEM((CB, width), jnp.float32) for _ in range(NBUF)]
            + [pltpu.SemaphoreType.DMA for _ in range(2 * NBUF)]
        ),
    )
    def scat(table_hbm, src_hbm, dst_hbm, zeros_hbm, out_hbm,
             sidx, didx, acc, *bufs_and_sems):
        rows = bufs_and_sems[:NBUF]
        gsem = bufs_and_sems[NBUF:2 * NBUF]
        ssem = bufs_and_sems[2 * NBUF:]
        c = lax.axis_index("c")
        s = lax.axis_index("s")
        wid = c * NS + s
        d0 = pltpu.async_copy(src_hbm.at[wid], sidx, gsem[0])
        d1 = pltpu.async_copy(dst_hbm.at[wid], didx, gsem[1])
        d2 = pltpu.async_copy(zeros_hbm.at[pl.ds(s * RPT, RPT)],
                              acc.at[pl.ds(s * RPT, RPT)], gsem[2])
        d0.wait()
        d1.wait()
        d2.wait()
        plsc.subcore_barrier()

        for b in range(NBUF):
            pltpu.async_copy(table_hbm.at[sidx.at[b]], rows[b], gsem[b])

        def body(t, carry):
            for b in range(NBUF):
                j = t * NBUF + b
                pltpu.make_async_copy(
                    table_hbm.at[sidx.at[j]], rows[b], gsem[b]).wait()
                pltpu.async_copy(rows[b], acc.at[didx.at[j]], ssem[b],
                                 add=True)
            for b in range(NBUF):
                j = t * NBUF + b
                pltpu.make_async_copy(
                    rows[b], acc.at[didx.at[j]], ssem[b]).wait()

                @pl.when(j + NBUF < CH)
                def _():
                    pltpu.async_copy(
                        table_hbm.at[sidx.at[j + NBUF]], rows[b], gsem[b])
            return carry

        lax.fori_loop(0, CH // NBUF, body, 0)
        plsc.subcore_barrier()
        pltpu.sync_copy(acc.at[pl.ds(s * RPT, RPT)],
                        out_hbm.at[c, pl.ds(s * RPT, RPT)])

    return scat


_scat16 = _make_edge_scatter(16)
_scat64 = _make_edge_scatter(64)
_scat32 = _make_edge_scatter(32)


def _make_deg_kernel():
    mesh = plsc.VectorSubcoreMesh(core_axis_name="c", subcore_axis_name="s")

    @functools.partial(
        pl.kernel,
        out_type=jax.ShapeDtypeStruct((NC, NP, DEGW), jnp.float32),
        mesh=mesh,
        compiler_params=pltpu.CompilerParams(use_tc_tiling_on_sc=False),
        scratch_types=[
            pltpu.VMEM((CH, CB), jnp.int32),
            pltpu.VMEM((CB, DEGW), jnp.float32),
            pltpu.VMEM_SHARED((NP, DEGW), jnp.float32),
            pltpu.SemaphoreType.DMA,
        ],
    )
    def degk(dst_hbm, ones_hbm, zeros_hbm, out_hbm, didx, ones_v, acc, sem):
        c = lax.axis_index("c")
        s = lax.axis_index("s")
        wid = c * NS + s
        d0 = pltpu.async_copy(dst_hbm.at[wid], didx, sem)
        pltpu.sync_copy(ones_hbm, ones_v)
        pltpu.sync_copy(zeros_hbm.at[pl.ds(s * RPT, RPT)],
                        acc.at[pl.ds(s * RPT, RPT)])
        d0.wait()
        plsc.subcore_barrier()

        def body(t, carry):
            for b in range(8):
                pltpu.async_copy(ones_v, acc.at[didx.at[t * 8 + b]], sem,
                                 add=True)
            for b in range(8):
                pltpu.make_async_copy(ones_v, acc.at[didx.at[t * 8 + b]],
                                      sem).wait()
            return carry

        lax.fori_loop(0, CH // 8, body, 0)
        plsc.subcore_barrier()
        pltpu.sync_copy(acc.at[pl.ds(s * RPT, RPT)],
                        out_hbm.at[c, pl.ds(s * RPT, RPT)])

    return degk


_degk = _make_deg_kernel()


def _tc1(xp, w1, degp):
    def body(x_ref, w_ref, d_ref, m_ref, dinv_ref):
        deg = d_ref[0][:, 0:1] + d_ref[1][:, 0:1] + 1.0
        dinv = lax.rsqrt(jnp.maximum(deg, 1.0))
        hw = jnp.dot(x_ref[...], w_ref[...], preferred_element_type=jnp.float32)
        m_ref[...] = dinv * hw
        dinv_ref[...] = dinv

    return pl.pallas_call(
        body,
        out_shape=(jax.ShapeDtypeStruct((NP, 16), jnp.float32),
                   jax.ShapeDtypeStruct((NP, 1), jnp.float32)),
    )(xp, w1, degp)


def _tc_mid(s_part, m, dinv, b, w_next, width_next):
    def body(s_ref, m_ref, dinv_ref, b_ref, w_ref, o_ref):
        h = dinv_ref[...] * (s_ref[0] + s_ref[1] + m_ref[...]) + b_ref[...]
        h = jnp.maximum(h, 0.0)
        o_ref[...] = dinv_ref[...] * jnp.dot(
            h, w_ref[...], preferred_element_type=jnp.float32)

    return pl.pallas_call(
        body,
        out_shape=jax.ShapeDtypeStruct((NP, width_next), jnp.float32),
    )(s_part, m, dinv, b, w_next)


def _tc_tail(s_part, m, dinv, b, batch_row, wa1, ba1, wa2, ba2):
    def body(s_ref, m_ref, dinv_ref, b_ref, batch_ref,
             wa1_ref, ba1_ref, wa2_ref, ba2_ref, o_ref):
        h3 = dinv_ref[...] * (s_ref[0] + s_ref[1] + m_ref[...]) + b_ref[...]
        iota = lax.broadcasted_iota(jnp.int32, (G, NP), 0)
        mask_t = (iota == batch_ref[...]).astype(jnp.float32)
        sums = jnp.dot(mask_t, h3, preferred_element_type=jnp.float32)
        cnt = jnp.sum(mask_t, axis=1, keepdims=True)
        pooled = sums / jnp.maximum(cnt, 1.0)
        a = jnp.dot(pooled, wa1_ref[...], preferred_element_type=jnp.float32)
        a = jnp.maximum(a + ba1_ref[...], 0.0)
        a = jnp.dot(a, wa2_ref[...], preferred_element_type=jnp.float32)
        o_ref[...] = jnp.tanh(a + ba2_ref[...])

    return pl.pallas_call(
        body,
        out_shape=jax.ShapeDtypeStruct((G, A_OUT), jnp.float32),
    )(s_part, m, dinv, b, batch_row, wa1, ba1, wa2, ba2)


def kernel(x, edge_index, batch, W1, b1, W2, b2, W3, b3, Wa1, ba1, Wa2, ba2):
    src = edge_index[0]
    dst = edge_index[1]
    pad = EP - E
    ar = jnp.arange(pad, dtype=jnp.int32)
    srcp = jnp.concatenate([src, ar % N]).reshape(NW, CH, CB)
    dstp = jnp.concatenate([dst, N + ar % (NP - N)]).reshape(NW, CH, CB)
    xp = jnp.zeros((NP, DF), jnp.float32).at[:N].set(x)
    batch_row = jnp.full((1, NP), G, jnp.int32).at[0, :N].set(batch)

    ones_deg = jnp.ones((CB, DEGW), jnp.float32)
    zdeg = jnp.zeros((NP, DEGW), jnp.float32)
    z16 = jnp.zeros((NP, 16), jnp.float32)
    z32 = jnp.zeros((NP, 32), jnp.float32)
    z64 = jnp.zeros((NP, 64), jnp.float32)

    degp = _degk(dstp, ones_deg, zdeg)
    m1, dinv = _tc1(xp, W1, degp)
    s1 = _scat16(m1, srcp, dstp, z16)
    m2 = _tc_mid(s1, m1, dinv, b1.reshape(1, 16), W2, 64)
    s2 = _scat64(m2, srcp, dstp, z64)
    m3 = _tc_mid(s2, m2, dinv, b2.reshape(1, 64), W3, 32)
    s3 = _scat32(m3, srcp, dstp, z32)
    return _tc_tail(s3, m3, dinv, b3.reshape(1, 32), batch_row,
                    Wa1, ba1.reshape(1, 64), Wa2, ba2.reshape(1, A_OUT))


# speedup vs baseline: 1.0292x; 1.0292x over previous
import functools

import jax
import jax.numpy as jnp
from jax import lax
from jax.experimental import pallas as pl
from jax.experimental.pallas import tpu as pltpu
from jax.experimental.pallas import tpu_sc as plsc

N = 10000
E = 320000
DF = 128
G = 64
A_OUT = 32

NC, NS = 2, 16
NW = NC * NS
CB = 128
NBUF = 8
CH = 80
EP = NW * CH * CB
NP = 10240
RPT = NP // NS
DEGW = 8


def _make_edge_scatter(width):
    mesh = plsc.VectorSubcoreMesh(core_axis_name="c", subcore_axis_name="s")

    @functools.partial(
        pl.kernel,
        out_type=jax.ShapeDtypeStruct((NC, NP, width), jnp.float32),
        mesh=mesh,
        compiler_params=pltpu.CompilerParams(use_tc_tiling_on_sc=False),
        scratch_types=(
            [pltpu.VMEM((CH, CB), jnp.int32),
             pltpu.VMEM((CH, CB), jnp.int32),
             pltpu.VMEM_SHARED((NP, width), jnp.float32)]
            + [pltpu.VMEM((CB, width), jnp.float32) for _ in range(NBUF)]
            + [pltpu.SemaphoreType.DMA for _ in range(2 * NBUF)]
        ),
    )
    def scat(table_hbm, src_hbm, dst_hbm, zeros_hbm, out_hbm,
             sidx, didx, acc, *bufs_and_sems):
        rows = bufs_and_sems[:NBUF]
        gsem = bufs_and_sems[NBUF:2 * NBUF]
        ssem = bufs_and_sems[2 * NBUF:]
        c = lax.axis_index("c")
        s = lax.axis_index("s")
        wid = c * NS + s
        d0 = pltpu.async_copy(src_hbm.at[wid], sidx, gsem[0])
        d1 = pltpu.async_copy(dst_hbm.at[wid], didx, gsem[1])
        d2 = pltpu.async_copy(zeros_hbm.at[pl.ds(s * RPT, RPT)],
                              acc.at[pl.ds(s * RPT, RPT)], gsem[2])
        d0.wait()
        d1.wait()
        d2.wait()
        plsc.subcore_barrier()

        for b in range(NBUF):
            pltpu.async_copy(table_hbm.at[sidx.at[b]], rows[b], gsem[b])

        def body(t, carry):
            for b in range(NBUF):
                j = t * NBUF + b
                pltpu.make_async_copy(
                    table_hbm.at[sidx.at[j]], rows[b], gsem[b]).wait()
                pltpu.async_copy(rows[b], acc.at[didx.at[j]], ssem[b],
                                 add=True)
            for b in range(NBUF):
                j = t * NBUF + b
                pltpu.make_async_copy(
                    rows[b], acc.at[didx.at[j]], ssem[b]).wait()

                @pl.when(j + NBUF < CH)
                def _():
                    pltpu.async_copy(
                        table_hbm.at[sidx.at[j + NBUF]], rows[b], gsem[b])
            return carry

        lax.fori_loop(0, CH // NBUF, body, 0)
        plsc.subcore_barrier()
        pltpu.sync_copy(acc.at[pl.ds(s * RPT, RPT)],
                        out_hbm.at[c, pl.ds(s * RPT, RPT)])

    return scat


_scat16 = _make_edge_scatter(16)
_scat64 = _make_edge_scatter(64)
_scat32 = _make_edge_scatter(32)


def _make_deg_kernel():
    mesh = plsc.VectorSubcoreMesh(core_axis_name="c", subcore_axis_name="s")

    @functools.partial(
        pl.kernel,
        out_type=jax.ShapeDtypeStruct((NC, NP, DEGW), jnp.float32),
        mesh=mesh,
        compiler_params=pltpu.CompilerParams(use_tc_tiling_on_sc=False),
        scratch_types=[
            pltpu.VMEM((CH, CB), jnp.int32),
            pltpu.VMEM((CB, DEGW), jnp.float32),
            pltpu.VMEM_SHARED((NP, DEGW), jnp.float32),
            pltpu.SemaphoreType.DMA,
        ],
    )
    def degk(dst_hbm, ones_hbm, zeros_hbm, out_hbm, didx, ones_v, acc, sem):
        c = lax.axis_index("c")
        s = lax.axis_index("s")
        wid = c * NS + s
        d0 = pltpu.async_copy(dst_hbm.at[wid], didx, sem)
        pltpu.sync_copy(ones_hbm, ones_v)
        pltpu.sync_copy(zeros_hbm.at[pl.ds(s * RPT, RPT)],
                        acc.at[pl.ds(s * RPT, RPT)])
        d0.wait()
        plsc.subcore_barrier()

        def body(t, carry):
            for b in range(8):
                pltpu.async_copy(ones_v, acc.at[didx.at[t * 8 + b]], sem,
                                 add=True)
            for b in range(8):
                pltpu.make_async_copy(ones_v, acc.at[didx.at[t * 8 + b]],
                                      sem).wait()
            return carry

        lax.fori_loop(0, CH // 8, body, 0)
        plsc.subcore_barrier()
        pltpu.sync_copy(acc.at[pl.ds(s * RPT, RPT)],
                        out_hbm.at[c, pl.ds(s * RPT, RPT)])

    return degk


_degk = _make_deg_kernel()


BN = 2048
NBLK = NP // BN


def _tc1(xp, w1, degp):
    def body(x_ref, w_ref, d_ref, m_ref, dinv_ref):
        deg = d_ref[0][:, 0:1] + d_ref[1][:, 0:1] + 1.0
        dinv = lax.rsqrt(jnp.maximum(deg, 1.0))
        hw = jnp.dot(x_ref[...], w_ref[...], preferred_element_type=jnp.float32)
        m_ref[...] = dinv * hw
        dinv_ref[...] = dinv

    return pl.pallas_call(
        body,
        grid=(NBLK,),
        in_specs=[pl.BlockSpec((BN, DF), lambda i: (i, 0)),
                  pl.BlockSpec((DF, 16), lambda i: (0, 0)),
                  pl.BlockSpec((NC, BN, DEGW), lambda i: (0, i, 0))],
        out_specs=(pl.BlockSpec((BN, 16), lambda i: (i, 0)),
                   pl.BlockSpec((BN, 1), lambda i: (i, 0))),
        out_shape=(jax.ShapeDtypeStruct((NP, 16), jnp.float32),
                   jax.ShapeDtypeStruct((NP, 1), jnp.float32)),
    )(xp, w1, degp)


def _tc_mid(s_part, m, dinv, b, w_next, width, width_next):
    def body(s_ref, m_ref, dinv_ref, b_ref, w_ref, o_ref):
        h = dinv_ref[...] * (s_ref[0] + s_ref[1] + m_ref[...]) + b_ref[...]
        h = jnp.maximum(h, 0.0)
        o_ref[...] = dinv_ref[...] * jnp.dot(
            h, w_ref[...], preferred_element_type=jnp.float32)

    return pl.pallas_call(
        body,
        grid=(NBLK,),
        in_specs=[pl.BlockSpec((NC, BN, width), lambda i: (0, i, 0)),
                  pl.BlockSpec((BN, width), lambda i: (i, 0)),
                  pl.BlockSpec((BN, 1), lambda i: (i, 0)),
                  pl.BlockSpec((1, width), lambda i: (0, 0)),
                  pl.BlockSpec((width, width_next), lambda i: (0, 0))],
        out_specs=pl.BlockSpec((BN, width_next), lambda i: (i, 0)),
        out_shape=jax.ShapeDtypeStruct((NP, width_next), jnp.float32),
    )(s_part, m, dinv, b, w_next)


def _tc_tail(s_part, m, dinv, b, batch_row, wa1, ba1, wa2, ba2):
    def body(s_ref, m_ref, dinv_ref, b_ref, batch_ref,
             wa1_ref, ba1_ref, wa2_ref, ba2_ref, o_ref, sums, cnt):
        i = pl.program_id(0)
        h3 = dinv_ref[...] * (s_ref[0] + s_ref[1] + m_ref[...]) + b_ref[...]
        iota = lax.broadcasted_iota(jnp.int32, (G, BN), 0)
        mask_t = (iota == batch_ref[...]).astype(jnp.float32)
        p_sums = jnp.dot(mask_t, h3, preferred_element_type=jnp.float32)
        p_cnt = jnp.sum(mask_t, axis=1, keepdims=True)

        @pl.when(i == 0)
        def _():
            sums[...] = p_sums
            cnt[...] = p_cnt

        @pl.when(i > 0)
        def _():
            sums[...] += p_sums
            cnt[...] += p_cnt

        @pl.when(i == NBLK - 1)
        def _():
            pooled = sums[...] / jnp.maximum(cnt[...], 1.0)
            a = jnp.dot(pooled, wa1_ref[...],
                        preferred_element_type=jnp.float32)
            a = jnp.maximum(a + ba1_ref[...], 0.0)
            a = jnp.dot(a, wa2_ref[...], preferred_element_type=jnp.float32)
            o_ref[...] = jnp.tanh(a + ba2_ref[...])

    return pl.pallas_call(
        body,
        grid=(NBLK,),
        in_specs=[pl.BlockSpec((NC, BN, A_OUT), lambda i: (0, i, 0)),
                  pl.BlockSpec((BN, A_OUT), lambda i: (i, 0)),
                  pl.BlockSpec((BN, 1), lambda i: (i, 0)),
                  pl.BlockSpec((1, A_OUT), lambda i: (0, 0)),
                  pl.BlockSpec((1, BN), lambda i: (0, i)),
                  pl.BlockSpec((A_OUT, G), lambda i: (0, 0)),
                  pl.BlockSpec((1, G), lambda i: (0, 0)),
                  pl.BlockSpec((G, A_OUT), lambda i: (0, 0)),
                  pl.BlockSpec((1, A_OUT), lambda i: (0, 0))],
        out_specs=pl.BlockSpec((G, A_OUT), lambda i: (0, 0)),
        out_shape=jax.ShapeDtypeStruct((G, A_OUT), jnp.float32),
        scratch_shapes=[pltpu.VMEM((G, A_OUT), jnp.float32),
                        pltpu.VMEM((G, 1), jnp.float32)],
    )(s_part, m, dinv, b, batch_row, wa1, ba1, wa2, ba2)


def kernel(x, edge_index, batch, W1, b1, W2, b2, W3, b3, Wa1, ba1, Wa2, ba2):
    src = edge_index[0]
    dst = edge_index[1]
    pad = EP - E
    ar = jnp.arange(pad, dtype=jnp.int32)
    srcp = jnp.concatenate([src, ar % N]).reshape(NW, CH, CB)
    dstp = jnp.concatenate([dst, N + ar % (NP - N)]).reshape(NW, CH, CB)
    xp = jnp.zeros((NP, DF), jnp.float32).at[:N].set(x)
    batch_row = jnp.full((1, NP), G, jnp.int32).at[0, :N].set(batch)

    ones_deg = jnp.ones((CB, DEGW), jnp.float32)
    zdeg = jnp.zeros((NP, DEGW), jnp.float32)
    z16 = jnp.zeros((NP, 16), jnp.float32)
    z32 = jnp.zeros((NP, 32), jnp.float32)
    z64 = jnp.zeros((NP, 64), jnp.float32)

    degp = _degk(dstp, ones_deg, zdeg)
    m1, dinv = _tc1(xp, W1, degp)
    s1 = _scat16(m1, srcp, dstp, z16)
    m2 = _tc_mid(s1, m1, dinv, b1.reshape(1, 16), W2, 16, 64)
    s2 = _scat64(m2, srcp, dstp, z64)
    m3 = _tc_mid(s2, m2, dinv, b2.reshape(1, 64), W3, 64, 32)
    s3 = _scat32(m3, srcp, dstp, z32)
    return _tc_tail(s3, m3, dinv, b3.reshape(1, 32), batch_row,
                    Wa1, ba1.reshape(1, 64), Wa2, ba2.reshape(1, A_OUT))
